# baseline (speedup 1.0000x reference)
import jax
import jax.numpy as jnp
from jax import lax
from jax.experimental import pallas as pl
from jax.experimental.pallas import tpu as pltpu
from jax.experimental.pallas import tpu_sc as plsc

N = 32768
NS = 16
L = 16
CHUNK = N // NS
VPL = CHUNK // L


def _argmax_body(values_hbm, out_hbm, vals_v, stage_v, stage_i,
                 shared_v, shared_i, merge_v, merge_i, out_v):
    sid = lax.axis_index("s")
    base = sid * CHUNK
    pltpu.sync_copy(values_hbm.at[pl.ds(base, CHUNK)], vals_v)

    lane = lax.iota(jnp.int32, 16)

    def step(j, carry):
        mv, mi = carry
        v = vals_v[pl.ds(j * L, L)]
        idx = base + j * L + lane
        p = v > mv
        return jnp.where(p, v, mv), jnp.where(p, idx, mi)

    mv0 = jnp.full((L,), -jnp.inf, jnp.float32)
    mi0 = jnp.zeros((L,), jnp.int32)
    mv, mi = lax.fori_loop(0, VPL, step, (mv0, mi0))

    stage_v[...] = mv
    stage_i[...] = mi
    pltpu.sync_copy(stage_v, shared_v.at[pl.ds(sid * L, L)])
    pltpu.sync_copy(stage_i, shared_i.at[pl.ds(sid * L, L)])
    plsc.subcore_barrier()

    @pl.when(sid == 0)
    def _():
        pltpu.sync_copy(shared_v, merge_v)
        pltpu.sync_copy(shared_i, merge_i)
        bmv = merge_v[pl.ds(0, L)]
        bmi = merge_i[pl.ds(0, L)]
        for t in range(1, NS):
            v = merge_v[pl.ds(t * L, L)]
            i = merge_i[pl.ds(t * L, L)]
            p = v > bmv
            bmv = jnp.where(p, v, bmv)
            bmi = jnp.where(p, i, bmi)
        for shift in (8, 4, 2, 1):
            perm = lax.bitwise_xor(lane, jnp.int32(shift))
            ov = bmv.at[perm].get(mode="promise_in_bounds")
            oi = bmi.at[perm].get(mode="promise_in_bounds")
            p = (ov > bmv) | ((ov == bmv) & (oi < bmi))
            bmv = jnp.where(p, ov, bmv)
            bmi = jnp.where(p, oi, bmi)
        out_v[...] = bmi
        pltpu.sync_copy(out_v, out_hbm)


_argmax_call = pl.kernel(
    _argmax_body,
    out_type=jax.ShapeDtypeStruct((L,), jnp.int32),
    mesh=plsc.VectorSubcoreMesh(
        core_axis_name="c", subcore_axis_name="s", num_cores=1),
    scratch_types=[
        pltpu.VMEM((CHUNK,), jnp.float32),
        pltpu.VMEM((L,), jnp.float32),
        pltpu.VMEM((L,), jnp.int32),
        pltpu.VMEM_SHARED((NS * L,), jnp.float32),
        pltpu.VMEM_SHARED((NS * L,), jnp.int32),
        pltpu.VMEM((NS * L,), jnp.float32),
        pltpu.VMEM((NS * L,), jnp.int32),
        pltpu.VMEM((L,), jnp.int32),
    ],
)


@jax.jit
def kernel(values, prefix_sum):
    out = _argmax_call(values)
    return out[0]


# speedup vs baseline: 1.0240x; 1.0240x over previous
import jax
import jax.numpy as jnp
from jax import lax
from jax.experimental import pallas as pl
from jax.experimental.pallas import tpu as pltpu
from jax.experimental.pallas import tpu_sc as plsc

N = 32768
NS = 16
L = 16
CHUNK = N // NS
VPL = CHUNK // L
UNROLL = 4


def _combine(av, ai, bv, bi):
    p = (bv > av) | ((bv == av) & (bi < ai))
    return jnp.where(p, bv, av), jnp.where(p, bi, ai)


def _argmax_body(values_hbm, out_hbm, vals_v, stage, shared, merge, out_v):
    sid = lax.axis_index("s")
    base = sid * CHUNK
    pltpu.sync_copy(values_hbm.at[pl.ds(base, CHUNK)], vals_v)

    lane = lax.iota(jnp.int32, 16)

    def step(j, carry):
        new = []
        for k in range(UNROLL):
            mv, mi = carry[k]
            off = (j * UNROLL + k) * L
            v = vals_v[pl.ds(off, L)]
            idx = base + off + lane
            p = v > mv
            new.append((jnp.where(p, v, mv), jnp.where(p, idx, mi)))
        return tuple(new)

    init = tuple(
        (jnp.full((L,), -jnp.inf, jnp.float32), jnp.zeros((L,), jnp.int32))
        for _ in range(UNROLL))
    accs = lax.fori_loop(0, VPL // UNROLL, step, init)
    (mv, mi) = accs[0]
    for k in range(1, UNROLL):
        mv, mi = _combine(mv, mi, accs[k][0], accs[k][1])

    stage[pl.ds(0, L)] = lax.bitcast_convert_type(mv, jnp.int32)
    stage[pl.ds(L, L)] = mi
    pltpu.sync_copy(stage, shared.at[pl.ds(sid * 2 * L, 2 * L)])
    plsc.subcore_barrier()

    @pl.when(sid == 0)
    def _():
        pltpu.sync_copy(shared, merge)
        bmv = lax.bitcast_convert_type(merge[pl.ds(0, L)], jnp.float32)
        bmi = merge[pl.ds(L, L)]
        for t in range(1, NS):
            v = lax.bitcast_convert_type(merge[pl.ds(t * 2 * L, L)], jnp.float32)
            i = merge[pl.ds(t * 2 * L + L, L)]
            bmv, bmi = _combine(bmv, bmi, v, i)
        for shift in (8, 4, 2, 1):
            perm = lax.bitwise_xor(lane, jnp.int32(shift))
            ov = bmv.at[perm].get(mode="promise_in_bounds")
            oi = bmi.at[perm].get(mode="promise_in_bounds")
            bmv, bmi = _combine(bmv, bmi, ov, oi)
        out_v[...] = bmi
        pltpu.sync_copy(out_v, out_hbm)


_argmax_call = pl.kernel(
    _argmax_body,
    out_type=jax.ShapeDtypeStruct((L,), jnp.int32),
    mesh=plsc.VectorSubcoreMesh(
        core_axis_name="c", subcore_axis_name="s", num_cores=1),
    scratch_types=[
        pltpu.VMEM((CHUNK,), jnp.float32),
        pltpu.VMEM((2 * L,), jnp.int32),
        pltpu.VMEM_SHARED((NS * 2 * L,), jnp.int32),
        pltpu.VMEM((NS * 2 * L,), jnp.int32),
        pltpu.VMEM((L,), jnp.int32),
    ],
)


@jax.jit
def kernel(values, prefix_sum):
    out = _argmax_call(values)
    return out[0]
